# baseline (speedup 1.0000x reference)
import functools

import jax
import jax.numpy as jnp
from jax import lax
from jax.experimental import pallas as pl
from jax.experimental.pallas import tpu as pltpu
from jax.experimental.pallas import tpu_sc as plsc

N = 10000
E = 160000
D = 256
H = 256

NC = 2
NS = 16
NW = NC * NS
N_PAD = 10240
ROWS_W = N_PAD // NW
IDXW = 80


BN = 1024


def _gather_x_body(feat_hbm, emb_hbm, x_hbm, idx_v, rows_v, sem):
    wid = lax.axis_index("s") * NC + lax.axis_index("c")
    base = wid * ROWS_W
    pltpu.sync_copy(feat_hbm.at[pl.ds(base, ROWS_W)], idx_v)
    cps = []
    for j in range(ROWS_W // IDXW):
        cps.append(pltpu.async_copy(
            emb_hbm.at[idx_v.at[pl.ds(j * IDXW, IDXW)]],
            rows_v.at[pl.ds(j * IDXW, IDXW)], sem))
    for cp in cps:
        cp.wait()
    pltpu.sync_copy(rows_v, x_hbm.at[pl.ds(base, ROWS_W)])


@functools.cache
def _gather_x_kernel():
    return pl.kernel(
        _gather_x_body,
        out_type=jax.ShapeDtypeStruct((N_PAD, D), jnp.float32),
        mesh=plsc.VectorSubcoreMesh(core_axis_name="c", subcore_axis_name="s"),
        scratch_types=[
            pltpu.VMEM((ROWS_W,), jnp.int32),
            pltpu.VMEM((ROWS_W, D), jnp.float32),
            pltpu.SemaphoreType.DMA,
        ],
    )


def _dense_body(x_ref, ms0_ref, ms1_ref, dg_ref,
                ws_ref, wn_ref, b_ref,
                we1_ref, be1_ref, we2_ref, be2_ref,
                wmu_ref, bmu_ref, wstd_ref, bstd_ref,
                we3_ref, be3_ref, we4_ref, be4_ref,
                wp_ref, bp_ref, noise_ref,
                y_ref, z_ref, mean_ref, ls_ref,
                acc_ref):
    i = pl.program_id(0)
    xb = x_ref[...]
    d = dg_ref[:, 0:1]
    inv = 1.0 / jnp.maximum(d, 1.0)
    nb = jnp.concatenate([ms0_ref[...], ms1_ref[...]], axis=1) * inv
    h = (jnp.dot(xb, ws_ref[...], preferred_element_type=jnp.float32)
         + jnp.dot(nb, wn_ref[...], preferred_element_type=jnp.float32)
         + b_ref[...])
    h = jnp.maximum(h, 0.0)
    rows = jax.lax.broadcasted_iota(jnp.int32, (BN, 1), 0) + i * BN
    h = jnp.where(rows < N, h, 0.0)
    part = jnp.sum(h, axis=0, keepdims=True)

    @pl.when(i == 0)
    def _():
        acc_ref[...] = part

    @pl.when(i > 0)
    def _():
        acc_ref[...] = acc_ref[...] + part

    @pl.when(i == pl.num_programs(0) - 1)
    def _():
        mm = lambda a, w: jnp.dot(a, w, preferred_element_type=jnp.float32)
        hg = acc_ref[...] * (1.0 / N)
        hidden = jnp.maximum(mm(hg, we1_ref[...]) + be1_ref[...], 0.0)
        h2 = jnp.maximum(mm(hidden, we2_ref[...]) + be2_ref[...], 0.0)
        mean = jnp.maximum(mm(h2, wmu_ref[...]) + bmu_ref[...], 0.0)
        log_std = jnp.maximum(mm(h2, wstd_ref[...]) + bstd_ref[...], 0.0)
        z = mean + noise_ref[...] * jnp.exp(0.5 * log_std)
        h3 = jnp.maximum(mm(z, we3_ref[...]) + be3_ref[...], 0.0)
        h4 = jnp.maximum(mm(h3, we4_ref[...]) + be4_ref[...], 0.0)
        y = mm(h4, wp_ref[...]) + bp_ref[...]
        y_ref[...] = y
        z_ref[...] = z
        mean_ref[...] = mean
        ls_ref[...] = log_std


def _dense(x, msum2, deg2, Ws, Wn, b, We1, be1, We2, be2, Wmu, bmu,
           Wstd, bstd, We3, be3, We4, be4, Wp, bp, noise, interpret=False):
    nblk = N_PAD // BN
    full = lambda shape: pl.BlockSpec(shape, lambda i: (0, 0))
    return pl.pallas_call(
        _dense_body,
        grid=(nblk,),
        in_specs=[
            pl.BlockSpec((BN, D), lambda i: (i, 0)),
            pl.BlockSpec((BN, D // 2), lambda i: (i, 0)),
            pl.BlockSpec((BN, D // 2), lambda i: (i + N_PAD // BN, 0)),
            pl.BlockSpec((BN, 16), lambda i: (i, 0)),
            full((D, D)), full((D, D)), full((1, D)),
            full((D, D)), full((1, D)), full((D, D)), full((1, D)),
            full((D, H)), full((1, H)), full((D, H)), full((1, H)),
            full((H, H)), full((1, H)), full((H, H)), full((1, H)),
            full((H, 1)), full((1, 1)), full((1, H)),
        ],
        out_specs=[
            full((1, 1)), full((1, H)), full((1, H)), full((1, H)),
        ],
        out_shape=[
            jax.ShapeDtypeStruct((1, 1), jnp.float32),
            jax.ShapeDtypeStruct((1, H), jnp.float32),
            jax.ShapeDtypeStruct((1, H), jnp.float32),
            jax.ShapeDtypeStruct((1, H), jnp.float32),
        ],
        scratch_shapes=[pltpu.VMEM((1, D), jnp.float32)],
        interpret=interpret,
    )(x, msum2, msum2, deg2, Ws, Wn, b, We1, be1, We2, be2, Wmu, bmu,
      Wstd, bstd, We3, be3, We4, be4, Wp, bp, noise)


def kernel(feat, edge_index, emb, Wself0, Wneigh0, b0, Wself1, Wneigh1, b1,
           We1, be1, We2, be2, Wmu, bmu, Wstd, bstd, We3, be3, We4, be4,
           Wp, bp, noise):
    feat_p = jnp.concatenate([feat, jnp.zeros((N_PAD - N,), jnp.int32)])
    src = edge_index[0]
    dst = edge_index[1]

    x = _gather_x_kernel()(feat_p, emb)
    msum = jax.ops.segment_sum(jnp.take(x, src, axis=0), dst,
                               num_segments=N_PAD)
    msum2 = jnp.concatenate([msum[:, :D // 2], msum[:, D // 2:]], axis=0)
    deg = jax.ops.segment_sum(jnp.ones((E,), jnp.float32), dst,
                              num_segments=N_PAD)
    deg2 = jnp.concatenate(
        [jnp.broadcast_to(deg[:, None], (N_PAD, 16))] * 2, axis=0)

    r = lambda v: v.reshape(1, -1)
    y, z, mean, log_std = _dense(
        x, msum2, deg2, Wself1, Wneigh1, r(b1),
        We1, r(be1), We2, r(be2), Wmu, r(bmu), Wstd, r(bstd),
        We3, r(be3), We4, r(be4), Wp, r(bp), noise)
    return (y, z, mean, log_std)
